# baseline (speedup 1.0000x reference)
import jax
import jax.numpy as jnp
from jax import lax
from jax.experimental import pallas as pl
from jax.experimental.pallas import tpu as pltpu
from jax.experimental.pallas import tpu_sc as plsc

_NUM_FIELDS = 26
_NUM_EMB = 100000
_EMB_DIM = 32
_BATCH = 4096
_FIELD_W = 4

_ROWS = _NUM_FIELDS * (_NUM_EMB + 1)
_G128 = (_ROWS * _EMB_DIM + 127) // 128
_NW = 32
_SPW = _BATCH // _NW
_S = 4
_NCH = _SPW // _S
_RPC = _S * _NUM_FIELDS * _FIELD_W
_GSZ = 104
_NG = _RPC // _GSZ
_GPC = _S * _NUM_FIELDS
_ODIM = _NUM_FIELDS * _EMB_DIM


def _body(gidx_hbm, soff_hbm, table_hbm, out_hbm, gidx_v, soff_v, rows_v,
          out_v, sem):
    wid = lax.axis_index("s") * 2 + lax.axis_index("c")

    def chunk(c, carry):
        row0 = (wid * _SPW + c * _S) * _NUM_FIELDS * _FIELD_W
        pltpu.sync_copy(gidx_hbm.at[pl.ds(row0, _RPC)], gidx_v)
        pltpu.sync_copy(soff_hbm.at[pl.ds(row0, _RPC)],
                        soff_v.at[pl.ds(0, _RPC)])
        cps = [
            pltpu.async_copy(table_hbm.at[gidx_v.at[pl.ds(j * _GSZ, _GSZ)]],
                             rows_v.at[pl.ds(j * _GSZ, _GSZ)], sem)
            for j in range(_NG)
        ]
        for cp in cps:
            cp.wait()

        def pool(g, carry):
            r = g * _FIELD_W
            offs4 = soff_v[pl.ds(r, 16)]
            o0 = offs4[0]
            o1 = offs4[1]
            o2 = offs4[2]
            o3 = offs4[3]
            for h in (0, 16):
                acc = (rows_v[r, pl.ds(o0 + h, 16)]
                       + rows_v[r + 1, pl.ds(o1 + h, 16)]
                       + rows_v[r + 2, pl.ds(o2 + h, 16)]
                       + rows_v[r + 3, pl.ds(o3 + h, 16)])
                out_v[pl.ds(g * _EMB_DIM + h, 16)] = acc * 0.25
            return carry

        lax.fori_loop(0, _GPC, pool, 0)
        pltpu.sync_copy(out_v,
                        out_hbm.at[pl.ds((wid * _SPW + c * _S) * _ODIM,
                                         _S * _ODIM)])
        return carry

    lax.fori_loop(0, _NCH, chunk, 0)


@jax.jit
def kernel(x, tables):
    x = x.astype(jnp.int32)
    offs = jnp.repeat(
        jnp.arange(_NUM_FIELDS, dtype=jnp.int32) * (_NUM_EMB + 1), _FIELD_W)
    r = (x + offs[None, :]).reshape(-1)
    gidx = lax.shift_right_logical(r, 2)
    soff = (r & 3) * _EMB_DIM

    flat = tables.reshape(-1)
    flat = jnp.pad(flat, (0, _G128 * 128 - flat.shape[0]))
    tab128 = flat.reshape(_G128, 128)

    k = pl.kernel(
        _body,
        out_type=jax.ShapeDtypeStruct((_BATCH * _ODIM,), jnp.float32),
        mesh=plsc.VectorSubcoreMesh(core_axis_name="c", subcore_axis_name="s"),
        scratch_types=[
            pltpu.VMEM((_RPC,), jnp.int32),
            pltpu.VMEM((_RPC + 16,), jnp.int32),
            pltpu.VMEM((_RPC, 128), jnp.float32),
            pltpu.VMEM((_S * _ODIM,), jnp.float32),
            pltpu.SemaphoreType.DMA,
        ],
    )
    return k(gidx, soff, tab128).reshape(_BATCH, _ODIM)


# speedup vs baseline: 5.5223x; 5.5223x over previous
import jax
import jax.numpy as jnp
from jax import lax
from jax.experimental import pallas as pl
from jax.experimental.pallas import tpu as pltpu
from jax.experimental.pallas import tpu_sc as plsc

_NUM_FIELDS = 26
_NUM_EMB = 100000
_EMB_DIM = 32
_BATCH = 4096
_FIELD_W = 4

_IPS = _NUM_FIELDS * _FIELD_W
_NW = 32
_SPW = _BATCH // _NW
_S = 4
_NCH = _SPW // _S
_RPC = _S * _IPS
_ODIM = _NUM_FIELDS * _EMB_DIM


def _body(x_hbm, table_hbm, out_hbm, idx_v, rows_v, out_v, sem):
    wid = lax.axis_index("s") * 2 + lax.axis_index("c")

    def chunk(c, carry):
        samp0 = wid * _SPW + c * _S
        pltpu.sync_copy(x_hbm.at[pl.ds(samp0 * _IPS, _RPC)],
                        idx_v.at[pl.ds(0, _RPC)])
        vecs = [idx_v[pl.ds(o, 16)] for o in range(0, _RPC, 16)]
        for j in range(_RPC):
            f = (j % _IPS) // _FIELD_W
            xi = vecs[j // 16][j % 16]
            pltpu.async_copy(table_hbm.at[f].at[pl.ds(xi, 1)],
                             rows_v.at[pl.ds(j, 1)], sem)
        pltpu.make_async_copy(table_hbm.at[0].at[pl.ds(0, _RPC)],
                              rows_v, sem).wait()

        def pool(q, carry):
            r = q * _FIELD_W
            for h in (0, 16):
                acc = (rows_v[r, pl.ds(h, 16)]
                       + rows_v[r + 1, pl.ds(h, 16)]
                       + rows_v[r + 2, pl.ds(h, 16)]
                       + rows_v[r + 3, pl.ds(h, 16)])
                out_v[pl.ds(q * _EMB_DIM + h, 16)] = acc * 0.25
            return carry

        lax.fori_loop(0, _S * _NUM_FIELDS, pool, 0)
        pltpu.sync_copy(out_v, out_hbm.at[pl.ds(samp0 * _ODIM, _S * _ODIM)])
        return carry

    lax.fori_loop(0, _NCH, chunk, 0)


@jax.jit
def kernel(x, tables):
    x = x.astype(jnp.int32).reshape(-1)
    k = pl.kernel(
        _body,
        out_type=jax.ShapeDtypeStruct((_BATCH * _ODIM,), jnp.float32),
        mesh=plsc.VectorSubcoreMesh(core_axis_name="c", subcore_axis_name="s"),
        scratch_types=[
            pltpu.VMEM((_RPC + 16,), jnp.int32),
            pltpu.VMEM((_RPC, _EMB_DIM), jnp.float32),
            pltpu.VMEM((_S * _ODIM,), jnp.float32),
            pltpu.SemaphoreType.DMA,
        ],
    )
    return k(x, tables).reshape(_BATCH, _ODIM)


# speedup vs baseline: 5.5347x; 1.0022x over previous
import jax
import jax.numpy as jnp
from jax import lax
from jax.experimental import pallas as pl
from jax.experimental.pallas import tpu as pltpu
from jax.experimental.pallas import tpu_sc as plsc

_NUM_FIELDS = 26
_NUM_EMB = 100000
_EMB_DIM = 32
_BATCH = 4096
_FIELD_W = 4

_IPS = _NUM_FIELDS * _FIELD_W
_NW = 32
_SPW = _BATCH // _NW
_S = 4
_NCH = _SPW // _S
_RPC = _S * _IPS
_ODIM = _NUM_FIELDS * _EMB_DIM


def _body(x_hbm, table_hbm, out_hbm, idx_v, rows_v, out_v, sem0, sem1, sem2,
          sem3):
    sems = (sem0, sem1, sem2, sem3)
    wid = lax.axis_index("s") * 2 + lax.axis_index("c")

    def chunk(c, carry):
        samp0 = wid * _SPW + c * _S
        pltpu.sync_copy(x_hbm.at[pl.ds(samp0 * _IPS, _RPC)],
                        idx_v.at[pl.ds(0, _RPC)])
        vecs = [idx_v[pl.ds(o, 16)] for o in range(0, _RPC, 16)]
        for j in range(_RPC):
            f = (j % _IPS) // _FIELD_W
            xi = vecs[j // 16][j % 16]
            pltpu.async_copy(table_hbm.at[f].at[pl.ds(xi, 1)],
                             rows_v.at[pl.ds(j, 1)], sems[j % 4])
        for q in range(4):
            pltpu.make_async_copy(table_hbm.at[0].at[pl.ds(0, _RPC // 4)],
                                  rows_v.at[pl.ds(0, _RPC // 4)],
                                  sems[q]).wait()

        def pool(q, carry):
            r = q * _FIELD_W
            for h in (0, 16):
                acc = (rows_v[r, pl.ds(h, 16)]
                       + rows_v[r + 1, pl.ds(h, 16)]
                       + rows_v[r + 2, pl.ds(h, 16)]
                       + rows_v[r + 3, pl.ds(h, 16)])
                out_v[pl.ds(q * _EMB_DIM + h, 16)] = acc * 0.25
            return carry

        lax.fori_loop(0, _S * _NUM_FIELDS, pool, 0)
        pltpu.sync_copy(out_v, out_hbm.at[pl.ds(samp0 * _ODIM, _S * _ODIM)])
        return carry

    lax.fori_loop(0, _NCH, chunk, 0)


@jax.jit
def kernel(x, tables):
    x = x.astype(jnp.int32).reshape(-1)
    k = pl.kernel(
        _body,
        out_type=jax.ShapeDtypeStruct((_BATCH * _ODIM,), jnp.float32),
        mesh=plsc.VectorSubcoreMesh(core_axis_name="c", subcore_axis_name="s"),
        scratch_types=[
            pltpu.VMEM((_RPC + 16,), jnp.int32),
            pltpu.VMEM((_RPC, _EMB_DIM), jnp.float32),
            pltpu.VMEM((_S * _ODIM,), jnp.float32),
            pltpu.SemaphoreType.DMA,
            pltpu.SemaphoreType.DMA,
            pltpu.SemaphoreType.DMA,
            pltpu.SemaphoreType.DMA,
        ],
    )
    return k(x, tables).reshape(_BATCH, _ODIM)


# speedup vs baseline: 5.7080x; 1.0313x over previous
import jax
import jax.numpy as jnp
from jax import lax
from jax.experimental import pallas as pl
from jax.experimental.pallas import tpu as pltpu
from jax.experimental.pallas import tpu_sc as plsc

_NUM_FIELDS = 26
_NUM_EMB = 100000
_EMB_DIM = 32
_BATCH = 4096
_FIELD_W = 4

_IPS = _NUM_FIELDS * _FIELD_W
_NW = 32
_SPW = _BATCH // _NW
_S = 4
_NCH = _SPW // _S
_RPC = _S * _IPS
_ODIM = _NUM_FIELDS * _EMB_DIM


def _body(x_hbm, table_hbm, out_hbm, idx_v, rows_v, out_v, sem0, sem1):
    wid = lax.axis_index("s") * 2 + lax.axis_index("c")
    sems = (sem0, sem1)

    def enqueue(c, par):
        samp0 = wid * _SPW + c * _S
        pltpu.sync_copy(x_hbm.at[pl.ds(samp0 * _IPS, _RPC)],
                        idx_v.at[pl.ds(0, _RPC)])
        vecs = [idx_v[pl.ds(o, 16)] for o in range(0, _RPC, 16)]
        for j in range(_RPC):
            f = (j % _IPS) // _FIELD_W
            xi = vecs[j // 16][j % 16]
            pltpu.async_copy(table_hbm.at[f].at[pl.ds(xi, 1)],
                             rows_v.at[par].at[pl.ds(j, 1)], sems[par])

    def drain_pool(c, par):
        pltpu.make_async_copy(table_hbm.at[0].at[pl.ds(0, _RPC)],
                              rows_v.at[par], sems[par]).wait()

        def pool(q, carry):
            r = q * _FIELD_W
            for h in (0, 16):
                acc = (rows_v[par, r, pl.ds(h, 16)]
                       + rows_v[par, r + 1, pl.ds(h, 16)]
                       + rows_v[par, r + 2, pl.ds(h, 16)]
                       + rows_v[par, r + 3, pl.ds(h, 16)])
                out_v[pl.ds(q * _EMB_DIM + h, 16)] = acc * 0.25
            return carry

        lax.fori_loop(0, _S * _NUM_FIELDS, pool, 0)
        pltpu.sync_copy(
            out_v,
            out_hbm.at[pl.ds((wid * _SPW + c * _S) * _ODIM, _S * _ODIM)])

    enqueue(0, 0)

    def step(k, carry):
        enqueue(2 * k + 1, 1)
        drain_pool(2 * k, 0)
        enqueue(2 * k + 2, 0)
        drain_pool(2 * k + 1, 1)
        return carry

    lax.fori_loop(0, _NCH // 2 - 1, step, 0)
    enqueue(_NCH - 1, 1)
    drain_pool(_NCH - 2, 0)
    drain_pool(_NCH - 1, 1)


@jax.jit
def kernel(x, tables):
    x = x.astype(jnp.int32).reshape(-1)
    k = pl.kernel(
        _body,
        out_type=jax.ShapeDtypeStruct((_BATCH * _ODIM,), jnp.float32),
        mesh=plsc.VectorSubcoreMesh(core_axis_name="c", subcore_axis_name="s"),
        scratch_types=[
            pltpu.VMEM((_RPC + 16,), jnp.int32),
            pltpu.VMEM((2, _RPC, _EMB_DIM), jnp.float32),
            pltpu.VMEM((_S * _ODIM,), jnp.float32),
            pltpu.SemaphoreType.DMA,
            pltpu.SemaphoreType.DMA,
        ],
    )
    return k(x, tables).reshape(_BATCH, _ODIM)
